# baseline (speedup 1.0000x reference)
import jax
import jax.numpy as jnp
from jax.experimental import pallas as pl
from jax.experimental.pallas import tpu as pltpu

_ROWS = 1000000
_DIM = 64
_CHUNK = 16384
_NFULL = _ROWS // _CHUNK
_TAILBASE = _NFULL * _CHUNK
_TAIL = _ROWS - _TAILBASE
_NBUF = 12
_DRAIN = 6


def _stream_kernel(in_hbm, out_hbm, buf, in_sems, out_sems):
    def in_copy(c, b):
        return pltpu.make_async_copy(
            in_hbm.at[:, pl.ds(c * _CHUNK, _CHUNK)],
            buf.at[b],
            in_sems.at[b],
        )

    def out_copy(c, b):
        return pltpu.make_async_copy(
            buf.at[b],
            out_hbm.at[:, pl.ds(c * _CHUNK, _CHUNK)],
            out_sems.at[b],
        )

    for c in range(_NBUF - _DRAIN):
        in_copy(c, c % _NBUF).start()
    for c in range(_NFULL):
        b = c % _NBUF
        in_copy(c, b).wait()
        out_copy(c, b).start()
        j = c - _DRAIN
        if j >= 0:
            out_copy(j, j % _NBUF).wait()
        nxt = c + _NBUF - _DRAIN
        if _NBUF - _DRAIN <= nxt < _NFULL:
            in_copy(nxt, nxt % _NBUF).start()
    for j in range(max(0, _NFULL - _DRAIN), _NFULL):
        out_copy(j, j % _NBUF).wait()


def kernel(weight):
    wt = weight.T
    out_t = pl.pallas_call(
        _stream_kernel,
        in_specs=[pl.BlockSpec(memory_space=pl.ANY)],
        out_specs=pl.BlockSpec(memory_space=pl.ANY),
        out_shape=jax.ShapeDtypeStruct((_DIM, _ROWS), jnp.float32),
        scratch_shapes=[
            pltpu.VMEM((_NBUF, _DIM, _CHUNK), jnp.float32),
            pltpu.SemaphoreType.DMA((_NBUF,)),
            pltpu.SemaphoreType.DMA((_NBUF,)),
        ],
    )(wt)
    return out_t.T


# speedup vs baseline: 1.0049x; 1.0049x over previous
import jax
import jax.numpy as jnp
from jax.experimental import pallas as pl
from jax.experimental.pallas import tpu as pltpu

_ROWS = 1000000
_DIM = 64
_BC = 49152


def _copy_block(in_ref, out_ref):
    out_ref[...] = in_ref[...]


def kernel(weight):
    wt = weight.T
    out_t = pl.pallas_call(
        _copy_block,
        grid=(pl.cdiv(_ROWS, _BC),),
        in_specs=[pl.BlockSpec((_DIM, _BC), lambda i: (0, i))],
        out_specs=pl.BlockSpec((_DIM, _BC), lambda i: (0, i)),
        out_shape=jax.ShapeDtypeStruct((_DIM, _ROWS), jnp.float32),
        compiler_params=pltpu.CompilerParams(
            dimension_semantics=("arbitrary",),
        ),
    )(wt)
    return out_t.T


# speedup vs baseline: 1.0061x; 1.0013x over previous
import jax
import jax.numpy as jnp
from jax.experimental import pallas as pl
from jax.experimental.pallas import tpu as pltpu

_ROWS = 1000000
_DIM = 64
_BC = 57344


def _copy_block(in_ref, out_ref):
    out_ref[...] = in_ref[...]


def kernel(weight):
    wt = weight.T
    out_t = pl.pallas_call(
        _copy_block,
        grid=(pl.cdiv(_ROWS, _BC),),
        in_specs=[pl.BlockSpec((_DIM, _BC), lambda i: (0, i))],
        out_specs=pl.BlockSpec((_DIM, _BC), lambda i: (0, i)),
        out_shape=jax.ShapeDtypeStruct((_DIM, _ROWS), jnp.float32),
        compiler_params=pltpu.CompilerParams(
            dimension_semantics=("arbitrary",),
        ),
    )(wt)
    return out_t.T
